# baseline (speedup 1.0000x reference)
import jax
import jax.numpy as jnp
from jax.experimental import pallas as pl
from jax.experimental.pallas import tpu as pltpu

B = 4096
D = 1024
H = 64
V = 2 ** 14

BLOCK_B = 256


def _body(x_ref, w1_ref, b1_ref, w2_ref, b2_ref, o_ref):
    w2f = w2_ref[:].astype(jnp.float32)
    cmax = jnp.sqrt(jnp.max(jnp.sum(w2f * w2f, axis=0)))
    bmax = jnp.max(jnp.abs(b2_ref[:]))

    h = jnp.maximum(
        jnp.dot(x_ref[:], w1_ref[:], preferred_element_type=jnp.float32)
        + b1_ref[:],
        0.0,
    )
    hn = jnp.sqrt(jnp.sum(h * h, axis=-1, keepdims=True))
    m = hn * cmax + bmax
    logits = jnp.dot(
        h.astype(jnp.bfloat16), w2_ref[:], preferred_element_type=jnp.float32
    )
    e = jnp.exp(logits + (b2_ref[:] - m))
    o_ref[:] = e
    s = jnp.sum(e, axis=-1, keepdims=True)
    o_ref[:] = o_ref[:] * (1.0 / s)


@jax.jit
def kernel(x_condition, W1, b1, W2, b2):
    b1r = b1.reshape(1, H)
    b2r = b2.reshape(1, V)
    W2 = W2.astype(jnp.bfloat16)
    grid = (B // BLOCK_B,)
    return pl.pallas_call(
        _body,
        grid=grid,
        in_specs=[
            pl.BlockSpec((BLOCK_B, D), lambda i: (i, 0)),
            pl.BlockSpec((D, H), lambda i: (0, 0)),
            pl.BlockSpec((1, H), lambda i: (0, 0)),
            pl.BlockSpec((H, V), lambda i: (0, 0)),
            pl.BlockSpec((1, V), lambda i: (0, 0)),
        ],
        out_specs=pl.BlockSpec((BLOCK_B, V), lambda i: (i, 0)),
        out_shape=jax.ShapeDtypeStruct((B, V), jnp.float32),
        compiler_params=pltpu.CompilerParams(
            dimension_semantics=("parallel",),
        ),
    )(x_condition, W1, b1r, W2, b2r)


# speedup vs baseline: 1.0551x; 1.0551x over previous
import jax
import jax.numpy as jnp
from jax.experimental import pallas as pl
from jax.experimental.pallas import tpu as pltpu

B = 4096
D = 1024
H = 64
V = 2 ** 14

BLOCK_B = 256


def _body(x_ref, w1_ref, b1_ref, w2_ref, b2_ref, o_ref, bound_ref):
    @pl.when(pl.program_id(0) == 0)
    def _():
        w2 = w2_ref[:]
        bound_ref[0, 0] = jnp.sqrt(jnp.max(jnp.sum(w2 * w2, axis=0)))
        bound_ref[0, 1] = jnp.max(jnp.abs(b2_ref[:]))

    h = jnp.maximum(
        jnp.dot(x_ref[:], w1_ref[:], preferred_element_type=jnp.float32)
        + b1_ref[:],
        0.0,
    )
    hn = jnp.sqrt(jnp.sum(h * h, axis=-1, keepdims=True))
    m = hn * bound_ref[0, 0] + bound_ref[0, 1]
    logits = jnp.dot(h, w2_ref[:], preferred_element_type=jnp.float32)
    e = jnp.exp(logits + (b2_ref[:] - m))
    o_ref[:] = e
    s = jnp.sum(e, axis=-1, keepdims=True)
    o_ref[:] = o_ref[:] * (1.0 / s)


@jax.jit
def kernel(x_condition, W1, b1, W2, b2):
    b1r = b1.reshape(1, H)
    b2r = b2.reshape(1, V)
    grid = (B // BLOCK_B,)
    return pl.pallas_call(
        _body,
        grid=grid,
        in_specs=[
            pl.BlockSpec((BLOCK_B, D), lambda i: (i, 0)),
            pl.BlockSpec((D, H), lambda i: (0, 0)),
            pl.BlockSpec((1, H), lambda i: (0, 0)),
            pl.BlockSpec((H, V), lambda i: (0, 0)),
            pl.BlockSpec((1, V), lambda i: (0, 0)),
        ],
        out_specs=pl.BlockSpec((BLOCK_B, V), lambda i: (i, 0)),
        out_shape=jax.ShapeDtypeStruct((B, V), jnp.float32),
        scratch_shapes=[pltpu.SMEM((1, 2), jnp.float32)],
        compiler_params=pltpu.CompilerParams(
            dimension_semantics=("arbitrary",),
        ),
    )(x_condition, W1, b1r, W2, b2r)
